# baseline (speedup 1.0000x reference)
import functools

import jax
import jax.numpy as jnp
import numpy as np
from jax import lax
from jax.experimental import pallas as pl
from jax.experimental.pallas import tpu as pltpu
from jax.experimental.pallas import tpu_sc as plsc

SPARSE_DIM = 16
GRID_DIM = 8
RES = SPARSE_DIM * GRID_DIM
N_PTS = 1048576

NC = 2
NS = 16
NW = NC * NS
L = 16

C = 1024
PER_W = N_PTS // NW
CHUNKS = PER_W // C

HI = np.float32(RES - 1.0 - 1e-6)
SCALE = np.float32(0.5 * RES)


def _axis_math(p):
    u_raw = (p + 1.0) * SCALE
    u = jnp.minimum(jnp.maximum(u_raw, 0.0), HI)
    b = u.astype(jnp.int32)
    f = u - b.astype(jnp.float32)
    inside = (u_raw > 0.0) & (u_raw < HI)
    edge = (u_raw == 0.0) | (u_raw == HI)
    gf = jnp.where(inside, SCALE, jnp.where(edge, np.float32(0.5) * SCALE, np.float32(0.0)))
    return b, f, gf


def _sc_body(px_hbm, py_hbm, pz_hbm, cube_hbm, emb_hbm, gx_hbm, gy_hbm, gz_hbm,
             px_v, py_v, pz_v,
             idx_a, idx_b, val_a, val_b,
             fx_a, fy_a, fz_a, gx_a, gy_a, gz_a,
             fx_b, fy_b, fz_b, gx_b, gy_b, gz_b,
             oe_v, ogx_v, ogy_v, ogz_v, sem_a, sem_b):
    bufs = (
        (idx_a, val_a, (fx_a, fy_a, fz_a), (gx_a, gy_a, gz_a), sem_a),
        (idx_b, val_b, (fx_b, fy_b, fz_b), (gx_b, gy_b, gz_b), sem_b),
    )
    wid = lax.axis_index("s") * NC + lax.axis_index("c")
    lane = lax.iota(jnp.int32, L)

    def stage(t, which):
        idx_v, val_v, fs, gs, sem = bufs[which]
        base = wid * PER_W + t * C
        pltpu.sync_copy(px_hbm.at[pl.ds(base, C)], px_v)
        pltpu.sync_copy(py_hbm.at[pl.ds(base, C)], py_v)
        pltpu.sync_copy(pz_hbm.at[pl.ds(base, C)], pz_v)

        def index_phase(i, carry):
            s = pl.ds(i * L, L)
            bx, fx, gfx = _axis_math(px_v[s])
            by, fy, gfy = _axis_math(py_v[s])
            bz, fz, gfz = _axis_math(pz_v[s])
            idx_v[s] = (bx << 14) | (by << 7) | bz
            fs[0][s] = fx
            fs[1][s] = fy
            fs[2][s] = fz
            gs[0][s] = gfx
            gs[1][s] = gfy
            gs[2][s] = gfz
            return carry

        lax.fori_loop(0, C // L, index_phase, 0)
        pltpu.async_copy(cube_hbm.at[idx_v], val_v, sem)

    def finish(t, which):
        idx_v, val_v, fs, gs, sem = bufs[which]
        pltpu.make_async_copy(cube_hbm.at[idx_v], val_v, sem).wait()

        def value_phase(i, carry):
            s = pl.ds(i * L, L)
            rows = (i * L + lane)
            v = [plsc.load_gather(val_v, [rows, jnp.full((L,), cc, jnp.int32)])
                 for cc in range(8)]
            fx, fy, fz = fs[0][s], fs[1][s], fs[2][s]
            gfx, gfy, gfz = gs[0][s], gs[1][s], gs[2][s]
            wz0, wz1 = 1.0 - fz, fz
            t00 = wz0 * v[0] + wz1 * v[1]
            t01 = wz0 * v[2] + wz1 * v[3]
            t10 = wz0 * v[4] + wz1 * v[5]
            t11 = wz0 * v[6] + wz1 * v[7]
            d00 = v[1] - v[0]
            d01 = v[3] - v[2]
            d10 = v[5] - v[4]
            d11 = v[7] - v[6]
            wy0, wy1 = 1.0 - fy, fy
            r0 = wy0 * t00 + wy1 * t01
            r1 = wy0 * t10 + wy1 * t11
            rz0 = wy0 * d00 + wy1 * d01
            rz1 = wy0 * d10 + wy1 * d11
            ry0 = t01 - t00
            ry1 = t11 - t10
            wx0, wx1 = 1.0 - fx, fx
            oe_v[s] = wx0 * r0 + wx1 * r1
            ogz_v[s] = gfz * (wx0 * rz0 + wx1 * rz1)
            ogy_v[s] = gfy * (wx0 * ry0 + wx1 * ry1)
            ogx_v[s] = gfx * (r1 - r0)
            return carry

        lax.fori_loop(0, C // L, value_phase, 0)
        base = wid * PER_W + t * C
        pltpu.sync_copy(oe_v, emb_hbm.at[pl.ds(base, C)])
        pltpu.sync_copy(ogx_v, gx_hbm.at[pl.ds(base, C)])
        pltpu.sync_copy(ogy_v, gy_hbm.at[pl.ds(base, C)])
        pltpu.sync_copy(ogz_v, gz_hbm.at[pl.ds(base, C)])

    stage(0, 0)

    def body(j, carry):
        t0 = 2 * j
        stage(t0 + 1, 1)
        finish(t0, 0)

        @pl.when(t0 + 2 < CHUNKS)
        def _():
            stage(t0 + 2, 0)

        finish(t0 + 1, 1)
        return carry


    lax.fori_loop(0, CHUNKS // 2, body, 0)


@jax.jit
def kernel(positions, table):
    pos_t = positions.T

    dense = table[:SPARSE_DIM ** 3, :, 0].reshape(
        SPARSE_DIM, SPARSE_DIM, SPARSE_DIM, GRID_DIM, GRID_DIM, GRID_DIM
    ).transpose(0, 3, 1, 4, 2, 5).reshape(RES, RES, RES)
    padded = jnp.pad(dense, ((0, 1), (0, 1), (0, 1)), mode="edge")
    cube = jnp.stack(
        [padded[a:a + RES, b:b + RES, c:c + RES]
         for a in (0, 1) for b in (0, 1) for c in (0, 1)],
        axis=-1,
    ).reshape(RES ** 3, 8)

    mesh = plsc.VectorSubcoreMesh(core_axis_name="c", subcore_axis_name="s")
    run = functools.partial(
        pl.kernel,
        mesh=mesh,
        compiler_params=pltpu.CompilerParams(
            needs_layout_passes=False, use_tc_tiling_on_sc=False),
        out_type=(
            jax.ShapeDtypeStruct((N_PTS,), jnp.float32),
            jax.ShapeDtypeStruct((N_PTS,), jnp.float32),
            jax.ShapeDtypeStruct((N_PTS,), jnp.float32),
            jax.ShapeDtypeStruct((N_PTS,), jnp.float32),
        ),
        scratch_types=(
            [pltpu.VMEM((C,), jnp.float32) for _ in range(3)]
            + [pltpu.VMEM((C,), jnp.int32), pltpu.VMEM((C,), jnp.int32)]
            + [pltpu.VMEM((C, 8), jnp.float32), pltpu.VMEM((C, 8), jnp.float32)]
            + [pltpu.VMEM((C,), jnp.float32) for _ in range(12)]
            + [pltpu.VMEM((C,), jnp.float32) for _ in range(4)]
            + [pltpu.SemaphoreType.DMA, pltpu.SemaphoreType.DMA]
        ),
    )(_sc_body)
    emb, gx, gy, gz = run(pos_t[0], pos_t[1], pos_t[2], cube)
    mask = jnp.all(jnp.abs(positions) <= 1.0, axis=-1)
    return emb[:, None], jnp.stack([gx, gy, gz], axis=-1), mask


# speedup vs baseline: 3.8608x; 3.8608x over previous
import functools

import jax
import jax.numpy as jnp
import numpy as np
from jax import lax
from jax.experimental import pallas as pl
from jax.experimental.pallas import tpu as pltpu
from jax.experimental.pallas import tpu_sc as plsc

SPARSE_DIM = 16
GRID_DIM = 8
RES = SPARSE_DIM * GRID_DIM
N_PTS = 1048576

NC = 2
NS = 16
NW = NC * NS
L = 16

C = 1024
PER_W = N_PTS // NW
CHUNKS = PER_W // C

HI = np.float32(RES - 1.0 - 1e-6)
SCALE = np.float32(0.5 * RES)


def _axis_math(p):
    u_raw = (p + 1.0) * SCALE
    u = jnp.minimum(jnp.maximum(u_raw, 0.0), HI)
    b = u.astype(jnp.int32)
    f = u - b.astype(jnp.float32)
    inside = (u_raw > 0.0) & (u_raw < HI)
    edge = (u_raw == 0.0) | (u_raw == HI)
    gf = jnp.where(inside, SCALE, jnp.where(edge, np.float32(0.5) * SCALE, np.float32(0.0)))
    return b, f, gf


def _sc_body(px_hbm, py_hbm, pz_hbm, flat_hbm, emb_hbm, gx_hbm, gy_hbm, gz_hbm,
             px_v, py_v, pz_v, *rest):
    idx_bufs = (rest[0:8], rest[8:16])
    val_bufs = (rest[16:24], rest[24:32])
    f_bufs = (rest[32:35], rest[35:38])
    g_bufs = (rest[38:41], rest[41:44])
    oe_v, ogx_v, ogy_v, ogz_v = rest[44:48]
    sems = rest[48:50]

    wid = lax.axis_index("s") * NC + lax.axis_index("c")

    def stage(t, which):
        idxs, vals, fs, gs, sem = idx_bufs[which], val_bufs[which], f_bufs[which], g_bufs[which], sems[which]
        base = wid * PER_W + t * C
        pltpu.sync_copy(px_hbm.at[pl.ds(base, C)], px_v)
        pltpu.sync_copy(py_hbm.at[pl.ds(base, C)], py_v)
        pltpu.sync_copy(pz_hbm.at[pl.ds(base, C)], pz_v)

        def index_phase(i, carry):
            s = pl.ds(i * L, L)
            bx, fx, gfx = _axis_math(px_v[s])
            by, fy, gfy = _axis_math(py_v[s])
            bz, fz, gfz = _axis_math(pz_v[s])
            x1 = jnp.minimum(bx + 1, RES - 1)
            y1 = jnp.minimum(by + 1, RES - 1)
            z1 = jnp.minimum(bz + 1, RES - 1)
            tx0 = (bx >> 3) << 17 | (bx & 7) << 6
            tx1 = (x1 >> 3) << 17 | (x1 & 7) << 6
            ty0 = (by >> 3) << 13 | (by & 7) << 3
            ty1 = (y1 >> 3) << 13 | (y1 & 7) << 3
            tz0 = (bz >> 3) << 9 | (bz & 7)
            tz1 = (z1 >> 3) << 9 | (z1 & 7)
            idxs[0][s] = tx0 | ty0 | tz0
            idxs[1][s] = tx0 | ty0 | tz1
            idxs[2][s] = tx0 | ty1 | tz0
            idxs[3][s] = tx0 | ty1 | tz1
            idxs[4][s] = tx1 | ty0 | tz0
            idxs[5][s] = tx1 | ty0 | tz1
            idxs[6][s] = tx1 | ty1 | tz0
            idxs[7][s] = tx1 | ty1 | tz1
            fs[0][s] = fx
            fs[1][s] = fy
            fs[2][s] = fz
            gs[0][s] = gfx
            gs[1][s] = gfy
            gs[2][s] = gfz
            return carry

        lax.fori_loop(0, C // L, index_phase, 0)
        for cc in range(8):
            pltpu.async_copy(flat_hbm.at[idxs[cc]], vals[cc], sem)

    def finish(t, which):
        idxs, vals, fs, gs, sem = idx_bufs[which], val_bufs[which], f_bufs[which], g_bufs[which], sems[which]
        for cc in range(8):
            pltpu.make_async_copy(flat_hbm.at[idxs[cc]], vals[cc], sem).wait()

        def value_phase(i, carry):
            s = pl.ds(i * L, L)
            fx, fy, fz = fs[0][s], fs[1][s], fs[2][s]
            gfx, gfy, gfz = gs[0][s], gs[1][s], gs[2][s]
            v = [vals[cc][s] for cc in range(8)]
            wz0, wz1 = 1.0 - fz, fz
            t00 = wz0 * v[0] + wz1 * v[1]
            t01 = wz0 * v[2] + wz1 * v[3]
            t10 = wz0 * v[4] + wz1 * v[5]
            t11 = wz0 * v[6] + wz1 * v[7]
            d00 = v[1] - v[0]
            d01 = v[3] - v[2]
            d10 = v[5] - v[4]
            d11 = v[7] - v[6]
            wy0, wy1 = 1.0 - fy, fy
            r0 = wy0 * t00 + wy1 * t01
            r1 = wy0 * t10 + wy1 * t11
            rz0 = wy0 * d00 + wy1 * d01
            rz1 = wy0 * d10 + wy1 * d11
            ry0 = t01 - t00
            ry1 = t11 - t10
            wx0, wx1 = 1.0 - fx, fx
            oe_v[s] = wx0 * r0 + wx1 * r1
            ogz_v[s] = gfz * (wx0 * rz0 + wx1 * rz1)
            ogy_v[s] = gfy * (wx0 * ry0 + wx1 * ry1)
            ogx_v[s] = gfx * (r1 - r0)
            return carry

        lax.fori_loop(0, C // L, value_phase, 0)
        base = wid * PER_W + t * C
        pltpu.sync_copy(oe_v, emb_hbm.at[pl.ds(base, C)])
        pltpu.sync_copy(ogx_v, gx_hbm.at[pl.ds(base, C)])
        pltpu.sync_copy(ogy_v, gy_hbm.at[pl.ds(base, C)])
        pltpu.sync_copy(ogz_v, gz_hbm.at[pl.ds(base, C)])

    stage(0, 0)

    def body(j, carry):
        t0 = 2 * j
        stage(t0 + 1, 1)
        finish(t0, 0)

        @pl.when(t0 + 2 < CHUNKS)
        def _():
            stage(t0 + 2, 0)

        finish(t0 + 1, 1)
        return carry

    lax.fori_loop(0, CHUNKS // 2, body, 0)


@jax.jit
def kernel(positions, table):
    pos_t = positions.T
    flat = table.reshape(-1)

    mesh = plsc.VectorSubcoreMesh(core_axis_name="c", subcore_axis_name="s")
    run = functools.partial(
        pl.kernel,
        mesh=mesh,
        out_type=(
            jax.ShapeDtypeStruct((N_PTS,), jnp.float32),
            jax.ShapeDtypeStruct((N_PTS,), jnp.float32),
            jax.ShapeDtypeStruct((N_PTS,), jnp.float32),
            jax.ShapeDtypeStruct((N_PTS,), jnp.float32),
        ),
        scratch_types=(
            [pltpu.VMEM((C,), jnp.float32) for _ in range(3)]
            + [pltpu.VMEM((C,), jnp.int32) for _ in range(16)]
            + [pltpu.VMEM((C,), jnp.float32) for _ in range(16)]
            + [pltpu.VMEM((C,), jnp.float32) for _ in range(12)]
            + [pltpu.VMEM((C,), jnp.float32) for _ in range(4)]
            + [pltpu.SemaphoreType.DMA, pltpu.SemaphoreType.DMA]
        ),
    )(_sc_body)
    emb, gx, gy, gz = run(pos_t[0], pos_t[1], pos_t[2], flat)
    mask = jnp.all(jnp.abs(positions) <= 1.0, axis=-1)
    return emb[:, None], jnp.stack([gx, gy, gz], axis=-1), mask


# speedup vs baseline: 3.9721x; 1.0288x over previous
import functools

import jax
import jax.numpy as jnp
import numpy as np
from jax import lax
from jax.experimental import pallas as pl
from jax.experimental.pallas import tpu as pltpu
from jax.experimental.pallas import tpu_sc as plsc

SPARSE_DIM = 16
GRID_DIM = 8
RES = SPARSE_DIM * GRID_DIM
N_PTS = 1048576

NC = 2
NS = 16
NW = NC * NS
L = 16

C = 1024
PER_W = N_PTS // NW
CHUNKS = PER_W // C

HI = np.float32(RES - 1.0 - 1e-6)
SCALE = np.float32(0.5 * RES)


def _axis_math(p):
    u_raw = (p + 1.0) * SCALE
    u = jnp.minimum(jnp.maximum(u_raw, 0.0), HI)
    b = u.astype(jnp.int32)
    f = u - b.astype(jnp.float32)
    inside = (u_raw > 0.0) & (u_raw < HI)
    edge = (u_raw == 0.0) | (u_raw == HI)
    gf = jnp.where(inside, SCALE, jnp.where(edge, np.float32(0.5) * SCALE, np.float32(0.0)))
    return b, f, gf


def _sc_body(px_hbm, py_hbm, pz_hbm, flat_hbm, emb_hbm, gx_hbm, gy_hbm, gz_hbm,
             *rest):
    pos_bufs = (rest[0:3], rest[3:6])
    idx_bufs = (rest[6:14], rest[14:22])
    val_bufs = (rest[22:30], rest[30:38])
    f_bufs = (rest[38:41], rest[41:44])
    g_bufs = (rest[44:47], rest[47:50])
    out_bufs = (rest[50:54], rest[54:58])
    gat_sems = rest[58:60]
    pos_sems = rest[60:62]
    out_sems = rest[62:64]

    wid = lax.axis_index("s") * NC + lax.axis_index("c")

    def fire_pos(t, which):
        ps, sem = pos_bufs[which], pos_sems[which]
        base = wid * PER_W + t * C
        pltpu.async_copy(px_hbm.at[pl.ds(base, C)], ps[0], sem)
        pltpu.async_copy(py_hbm.at[pl.ds(base, C)], ps[1], sem)
        pltpu.async_copy(pz_hbm.at[pl.ds(base, C)], ps[2], sem)

    def wait_pos(which):
        ps, sem = pos_bufs[which], pos_sems[which]
        base0 = wid * PER_W
        pltpu.make_async_copy(px_hbm.at[pl.ds(base0, C)], ps[0], sem).wait()
        pltpu.make_async_copy(py_hbm.at[pl.ds(base0, C)], ps[1], sem).wait()
        pltpu.make_async_copy(pz_hbm.at[pl.ds(base0, C)], ps[2], sem).wait()

    def stage(t, which):
        ps = pos_bufs[which]
        idxs, fs, gs = idx_bufs[which], f_bufs[which], g_bufs[which]
        vals, sem = val_bufs[which], gat_sems[which]
        wait_pos(which)

        def index_phase(i, carry):
            s = pl.ds(i * L, L)
            bx, fx, gfx = _axis_math(ps[0][s])
            by, fy, gfy = _axis_math(ps[1][s])
            bz, fz, gfz = _axis_math(ps[2][s])
            x1 = jnp.minimum(bx + 1, RES - 1)
            y1 = jnp.minimum(by + 1, RES - 1)
            z1 = jnp.minimum(bz + 1, RES - 1)
            tx0 = (bx >> 3) << 17 | (bx & 7) << 6
            tx1 = (x1 >> 3) << 17 | (x1 & 7) << 6
            ty0 = (by >> 3) << 13 | (by & 7) << 3
            ty1 = (y1 >> 3) << 13 | (y1 & 7) << 3
            tz0 = (bz >> 3) << 9 | (bz & 7)
            tz1 = (z1 >> 3) << 9 | (z1 & 7)
            idxs[0][s] = tx0 | ty0 | tz0
            idxs[1][s] = tx0 | ty0 | tz1
            idxs[2][s] = tx0 | ty1 | tz0
            idxs[3][s] = tx0 | ty1 | tz1
            idxs[4][s] = tx1 | ty0 | tz0
            idxs[5][s] = tx1 | ty0 | tz1
            idxs[6][s] = tx1 | ty1 | tz0
            idxs[7][s] = tx1 | ty1 | tz1
            fs[0][s] = fx
            fs[1][s] = fy
            fs[2][s] = fz
            gs[0][s] = gfx
            gs[1][s] = gfy
            gs[2][s] = gfz
            return carry

        lax.fori_loop(0, C // L, index_phase, 0)
        for cc in range(8):
            pltpu.async_copy(flat_hbm.at[idxs[cc]], vals[cc], sem)

        @pl.when(t + 2 < CHUNKS)
        def _():
            fire_pos(t + 2, which)

    def wait_out(which):
        ob, sem = out_bufs[which], out_sems[which]
        base0 = wid * PER_W
        pltpu.make_async_copy(ob[0], emb_hbm.at[pl.ds(base0, C)], sem).wait()
        pltpu.make_async_copy(ob[1], gx_hbm.at[pl.ds(base0, C)], sem).wait()
        pltpu.make_async_copy(ob[2], gy_hbm.at[pl.ds(base0, C)], sem).wait()
        pltpu.make_async_copy(ob[3], gz_hbm.at[pl.ds(base0, C)], sem).wait()

    def finish(t, which):
        idxs, vals, sem = idx_bufs[which], val_bufs[which], gat_sems[which]
        fs, gs = f_bufs[which], g_bufs[which]
        ob, osem = out_bufs[which], out_sems[which]
        for cc in range(8):
            pltpu.make_async_copy(flat_hbm.at[idxs[cc]], vals[cc], sem).wait()

        @pl.when(t >= 2)
        def _():
            wait_out(which)

        def value_phase(i, carry):
            s = pl.ds(i * L, L)
            fx, fy, fz = fs[0][s], fs[1][s], fs[2][s]
            gfx, gfy, gfz = gs[0][s], gs[1][s], gs[2][s]
            v = [vals[cc][s] for cc in range(8)]
            wz0, wz1 = 1.0 - fz, fz
            t00 = wz0 * v[0] + wz1 * v[1]
            t01 = wz0 * v[2] + wz1 * v[3]
            t10 = wz0 * v[4] + wz1 * v[5]
            t11 = wz0 * v[6] + wz1 * v[7]
            d00 = v[1] - v[0]
            d01 = v[3] - v[2]
            d10 = v[5] - v[4]
            d11 = v[7] - v[6]
            wy0, wy1 = 1.0 - fy, fy
            r0 = wy0 * t00 + wy1 * t01
            r1 = wy0 * t10 + wy1 * t11
            rz0 = wy0 * d00 + wy1 * d01
            rz1 = wy0 * d10 + wy1 * d11
            ry0 = t01 - t00
            ry1 = t11 - t10
            wx0, wx1 = 1.0 - fx, fx
            ob[0][s] = wx0 * r0 + wx1 * r1
            ob[3][s] = gfz * (wx0 * rz0 + wx1 * rz1)
            ob[2][s] = gfy * (wx0 * ry0 + wx1 * ry1)
            ob[1][s] = gfx * (r1 - r0)
            return carry

        lax.fori_loop(0, C // L, value_phase, 0)
        base = wid * PER_W + t * C
        pltpu.async_copy(ob[0], emb_hbm.at[pl.ds(base, C)], osem)
        pltpu.async_copy(ob[1], gx_hbm.at[pl.ds(base, C)], osem)
        pltpu.async_copy(ob[2], gy_hbm.at[pl.ds(base, C)], osem)
        pltpu.async_copy(ob[3], gz_hbm.at[pl.ds(base, C)], osem)

    fire_pos(0, 0)
    fire_pos(1, 1)
    stage(0, 0)

    def body(j, carry):
        t0 = 2 * j
        stage(t0 + 1, 1)
        finish(t0, 0)

        @pl.when(t0 + 2 < CHUNKS)
        def _():
            stage(t0 + 2, 0)

        finish(t0 + 1, 1)
        return carry

    lax.fori_loop(0, CHUNKS // 2, body, 0)
    wait_out(0)
    wait_out(1)


@jax.jit
def kernel(positions, table):
    pos_t = positions.T
    flat = table.reshape(-1)

    mesh = plsc.VectorSubcoreMesh(core_axis_name="c", subcore_axis_name="s")
    run = functools.partial(
        pl.kernel,
        mesh=mesh,
        out_type=(
            jax.ShapeDtypeStruct((N_PTS,), jnp.float32),
            jax.ShapeDtypeStruct((N_PTS,), jnp.float32),
            jax.ShapeDtypeStruct((N_PTS,), jnp.float32),
            jax.ShapeDtypeStruct((N_PTS,), jnp.float32),
        ),
        scratch_types=(
            [pltpu.VMEM((C,), jnp.float32) for _ in range(6)]
            + [pltpu.VMEM((C,), jnp.int32) for _ in range(16)]
            + [pltpu.VMEM((C,), jnp.float32) for _ in range(16)]
            + [pltpu.VMEM((C,), jnp.float32) for _ in range(12)]
            + [pltpu.VMEM((C,), jnp.float32) for _ in range(8)]
            + [pltpu.SemaphoreType.DMA for _ in range(6)]
        ),
    )(_sc_body)
    emb, gx, gy, gz = run(pos_t[0], pos_t[1], pos_t[2], flat)
    mask = jnp.all(jnp.abs(positions) <= 1.0, axis=-1)
    return emb[:, None], jnp.stack([gx, gy, gz], axis=-1), mask
